# baseline (speedup 1.0000x reference)
import jax
import jax.numpy as jnp
from jax import lax
from jax.experimental import pallas as pl
from jax.experimental.pallas import tpu as pltpu
from jax.experimental.pallas import tpu_sc as plsc

N = 10000
E = 320000
D = 128

NC = 2
NS = 16
NW = NC * NS
CH = 128
NCH = -(-E // (NW * CH))
EPW = NCH * CH
E_PAD = EPW * NW
ACC_ROWS = N + 8
ST = 640

_MB = 1000


def _agg_body(src_hbm, dst_hbm, m_hbm, out_hbm,
              src0, src1, src2, dst0, dst1, dst2, r0, r1, r2, acc,
              si0, si1, si2, sg0, sg1, sg2, ss0, ss1, ss2):
    srcv = [src0, src1, src2]
    dstv = [dst0, dst1, dst2]
    rows = [r0, r1, r2]
    semi = [si0, si1, si2]
    semg = [sg0, sg1, sg2]
    sems = [ss0, ss1, ss2]
    c = lax.axis_index("c")
    s = lax.axis_index("s")

    zero16 = jnp.zeros((16,), jnp.float32)

    def _zfill(i, carry):
        for j in range(8):
            r0[i, pl.ds(j * 16, 16)] = zero16
        return carry

    lax.fori_loop(0, CH, _zfill, 0)
    ofs = pl.multiple_of(s * ST, 8)

    @pl.when(s < 15)
    def _():
        for k in range(ST // CH):
            pltpu.sync_copy(r0, acc.at[pl.ds(ofs + k * CH, CH), :])

    @pl.when(s == 15)
    def _():
        for k in range(3):
            pltpu.sync_copy(r0, acc.at[pl.ds(15 * ST + k * CH, CH), :])
        pltpu.sync_copy(r0.at[pl.ds(0, 24), :],
                        acc.at[pl.ds(15 * ST + 3 * CH, 24), :])

    plsc.subcore_barrier()

    ebase = pl.multiple_of((c * NS + s) * EPW, 8)

    def fetch(i, b):
        base = pl.multiple_of(ebase + i * CH, 8)
        pltpu.async_copy(src_hbm.at[pl.ds(base, CH)], srcv[b], semi[b])
        pltpu.async_copy(dst_hbm.at[pl.ds(base, CH)], dstv[b], semi[b])

    def wait_idx(b):
        pltpu.make_async_copy(src_hbm.at[pl.ds(0, CH)], srcv[b],
                              semi[b]).wait()
        pltpu.make_async_copy(dst_hbm.at[pl.ds(0, CH)], dstv[b],
                              semi[b]).wait()

    def gather(b):
        pltpu.async_copy(m_hbm.at[srcv[b]], rows[b], semg[b])

    def wait_gather(b):
        pltpu.make_async_copy(m_hbm.at[pl.ds(0, CH), :], rows[b],
                              semg[b]).wait()

    def scatter(b):
        pltpu.async_copy(rows[b], acc.at[dstv[b]], sems[b], add=True)

    def wait_scatter(b):
        pltpu.make_async_copy(rows[b], acc.at[pl.ds(0, CH), :],
                              sems[b]).wait()

    fetch(0, 0)
    fetch(1, 1)
    wait_idx(0)
    gather(0)

    def _slot(j, carry):
        for b in range(3):
            i = 3 * j + b
            bf = (b + 2) % 3
            bg = (b + 1) % 3

            @pl.when(i + 2 < NCH)
            def _():
                @pl.when(i >= 1)
                def _():
                    wait_scatter(bf)

                fetch(i + 2, bf)

            @pl.when(i + 1 < NCH)
            def _():
                wait_idx(bg)
                gather(bg)

            @pl.when(i < NCH)
            def _():
                wait_gather(b)
                scatter(b)

        return carry

    lax.fori_loop(0, -(-NCH // 3), _slot, 0)
    for b in range(3):
        wait_scatter(b)

    plsc.subcore_barrier()

    @pl.when(s < 15)
    def _():
        pltpu.sync_copy(acc.at[pl.ds(ofs, ST), :],
                        out_hbm.at[c, pl.ds(ofs, ST), :])

    @pl.when(s == 15)
    def _():
        pltpu.sync_copy(acc.at[pl.ds(15 * ST, N - 15 * ST), :],
                        out_hbm.at[c, pl.ds(15 * ST, N - 15 * ST), :])


@jax.jit
def _agg(src, dst, m):
    mesh = plsc.VectorSubcoreMesh(core_axis_name="c", subcore_axis_name="s")
    idx_t = pltpu.VMEM((CH,), jnp.int32)
    row_t = pltpu.VMEM((CH, D), jnp.float32)
    return pl.kernel(
        _agg_body,
        out_type=jax.ShapeDtypeStruct((NC, N, D), jnp.float32),
        mesh=mesh,
        scratch_types=[
            idx_t, idx_t, idx_t, idx_t, idx_t, idx_t,
            row_t, row_t, row_t,
            pltpu.VMEM_SHARED((ACC_ROWS, D), jnp.float32),
        ] + [pltpu.SemaphoreType.DMA] * 9,
    )(src, dst, m)


def _mm_body(x_ref, w_ref, o_ref):
    o_ref[...] = jnp.dot(x_ref[...], w_ref[...],
                         preferred_element_type=jnp.float32)


@jax.jit
def _mm(x, w):
    return pl.pallas_call(
        _mm_body,
        grid=(N // _MB,),
        in_specs=[
            pl.BlockSpec((_MB, D), lambda i: (i, 0)),
            pl.BlockSpec((D, D), lambda i: (0, 0)),
        ],
        out_specs=pl.BlockSpec((_MB, D), lambda i: (i, 0)),
        out_shape=jax.ShapeDtypeStruct((N, D), jnp.float32),
    )(x, w)


def _combine_mm_body(p_ref, b_ref, w_ref, o_ref):
    x = jnp.maximum(p_ref[0] + p_ref[1] + b_ref[...], 0.0)
    o_ref[...] = jnp.dot(x, w_ref[...], preferred_element_type=jnp.float32)


@jax.jit
def _combine_mm(p, b, w):
    return pl.pallas_call(
        _combine_mm_body,
        grid=(N // _MB,),
        in_specs=[
            pl.BlockSpec((NC, _MB, D), lambda i: (0, i, 0)),
            pl.BlockSpec((1, D), lambda i: (0, 0)),
            pl.BlockSpec((D, D), lambda i: (0, 0)),
        ],
        out_specs=pl.BlockSpec((_MB, D), lambda i: (i, 0)),
        out_shape=jax.ShapeDtypeStruct((N, D), jnp.float32),
    )(p, b, w)


def _head_body(p_ref, b2_ref, wh1_ref, bh1_ref, wh2_ref, bh2_ref,
               wc_ref, bc_ref, o_ref):
    x = jnp.maximum(p_ref[0] + p_ref[1] + b2_ref[...], 0.0)
    h = jnp.maximum(
        jnp.dot(x, wh1_ref[...], preferred_element_type=jnp.float32)
        + bh1_ref[...], 0.0)
    h = jnp.maximum(
        jnp.dot(h, wh2_ref[...], preferred_element_type=jnp.float32)
        + bh2_ref[...], 0.0)
    av = (jnp.dot(h, wc_ref[...], preferred_element_type=jnp.float32)
          + bc_ref[...])
    col = lax.broadcasted_iota(jnp.int32, av.shape, 1)
    adv_sum = jnp.sum(jnp.where(col < 5, av, 0.0), axis=1, keepdims=True)
    val = jnp.sum(jnp.where(col == 5, av, 0.0), axis=1, keepdims=True)
    o_ref[...] = val + av - adv_sum * (1.0 / 5.0)


@jax.jit
def _head(p, b2, wh1, bh1, wh2, bh2, wc, bc):
    return pl.pallas_call(
        _head_body,
        grid=(N // _MB,),
        in_specs=[
            pl.BlockSpec((NC, _MB, D), lambda i: (0, i, 0)),
            pl.BlockSpec((1, D), lambda i: (0, 0)),
            pl.BlockSpec((D, D), lambda i: (0, 0)),
            pl.BlockSpec((1, D), lambda i: (0, 0)),
            pl.BlockSpec((D, D), lambda i: (0, 0)),
            pl.BlockSpec((1, D), lambda i: (0, 0)),
            pl.BlockSpec((D, 8), lambda i: (0, 0)),
            pl.BlockSpec((1, 8), lambda i: (0, 0)),
        ],
        out_specs=pl.BlockSpec((_MB, 8), lambda i: (i, 0)),
        out_shape=jax.ShapeDtypeStruct((N, 8), jnp.float32),
    )(p, b2, wh1, bh1, wh2, bh2, wc, bc)


def kernel(edge_index, entity_embeddings, W1, b1, W2, b2,
           Wh1, bh1, Wh2, bh2, Wadv, badv, Wval, bval):
    pad = E_PAD - E
    src = jnp.concatenate([edge_index[0], jnp.zeros((pad,), jnp.int32)])
    dst = jnp.concatenate([edge_index[1], jnp.full((pad,), N, jnp.int32)])
    wc = jnp.concatenate([Wadv, Wval, jnp.zeros((D, 2), jnp.float32)], axis=1)
    bc = jnp.concatenate([badv, bval, jnp.zeros((2,), jnp.float32)])[None, :]

    m1 = _mm(entity_embeddings, W1)
    p1 = _agg(src, dst, m1)
    m2 = _combine_mm(p1, b1[None, :], W2)
    p2 = _agg(src, dst, m2)
    q8 = _head(p2, b2[None, :], Wh1, bh1[None, :], Wh2, bh2[None, :], wc, bc)
    return q8[:, :5]
